# baseline (speedup 1.0000x reference)
import jax
import jax.numpy as jnp
from jax import lax
from jax.experimental import pallas as pl
from jax.experimental.pallas import tpu as pltpu
from jax.experimental.pallas import tpu_sc as plsc

VOCAB = 100000
D = 128
W = 200
B = 1024

NC, NS = 2, 16
NW = NC * NS
ROWS_PER_W = B // NW
H = 2
WH = W // H
NBUF = 3
L = 16
RU = 2


def _body(tab_hbm, tok_hbm, pos_hbm, out_hbm, idx_v, pos_v, *scratch):
    bufs = list(scratch[:NBUF])
    sem_idx = scratch[NBUF]
    sem_pv = scratch[NBUF + 1]
    sem_gat = list(scratch[NBUF + 2:NBUF + 2 + NBUF])
    sem_out = list(scratch[NBUF + 2 + NBUF:NBUF + 2 + 2 * NBUF])

    sid = lax.axis_index("s")
    wid = lax.axis_index("c") * NS + sid

    d_idx = pltpu.async_copy(tok_hbm.at[pl.ds(wid * ROWS_PER_W, ROWS_PER_W)],
                             idx_v, sem_idx)
    d_pv = pltpu.async_copy(pos_hbm, pos_v, sem_pv)

    d_gat = [None] * NBUF
    d_out = [None] * NBUF

    def start_gather(jw):
        s = jw % NBUF
        if d_out[s] is not None:
            d_out[s].wait()
            d_out[s] = None
        d_gat[s] = [
            pltpu.async_copy(
                tab_hbm.at[idx_v.at[jw].at[h]],
                bufs[s].at[pl.ds(h * WH, WH)],
                sem_gat[s])
            for h in range(H)
        ]

    def finish(jw):
        s = jw % NBUF
        for d in d_gat[s]:
            d.wait()
        buf = bufs[s]

        def add_rows(i, carry):
            for r in range(RU):
                for k in range(D // L):
                    sl = pl.ds(k * L, L)
                    plsc.addupdate(buf.at[i * RU + r, sl],
                                   pos_v[i * RU + r, sl])
            return carry

        lax.fori_loop(0, W // RU, add_rows, 0, unroll=2)
        row = wid * ROWS_PER_W + jw
        d_out[s] = pltpu.async_copy(buf, out_hbm.at[pl.ds(row * W, W)],
                                    sem_out[s])

    d_idx.wait()
    d_pv.wait()
    start_gather(0)
    start_gather(1)

    for j in range(ROWS_PER_W):
        if j + 2 < ROWS_PER_W:
            start_gather(j + 2)
        finish(j)

    for s in range(NBUF):
        if d_out[s] is not None:
            d_out[s].wait()


def kernel(tokens, token_embedding, position_embedding, position_indices):
    del position_indices
    tokens3 = tokens.reshape(B, H, WH).astype(jnp.int32)
    mesh = plsc.VectorSubcoreMesh(
        core_axis_name="c", subcore_axis_name="s",
        num_cores=NC, num_subcores=NS,
    )
    out = pl.kernel(
        _body,
        out_type=jax.ShapeDtypeStruct((B * W, D), jnp.float32),
        mesh=mesh,
        scratch_types=[
            pltpu.VMEM((ROWS_PER_W, H, WH), jnp.int32),
            pltpu.VMEM((W, D), jnp.float32),
        ] + [pltpu.VMEM((W, D), jnp.float32)] * NBUF
          + [pltpu.SemaphoreType.DMA] * (2 + 2 * NBUF),
    )(token_embedding, tokens3, position_embedding)
    return out.reshape(B, W, D)


# speedup vs baseline: 1.2722x; 1.2722x over previous
import jax
import jax.numpy as jnp
from jax import lax
from jax.experimental import pallas as pl
from jax.experimental.pallas import tpu as pltpu
from jax.experimental.pallas import tpu_sc as plsc

VOCAB = 100000
D = 128
W = 200
B = 1024

NC, NS = 2, 16
NW = NC * NS
ROWS_PER_W = B // NW
H = 2
WH = W // H
NBUF = 4


def _body(tab_hbm, tok_hbm, pos_hbm, out_hbm, idx_v, pos_s, *scratch):
    bufs = list(scratch[:NBUF])
    sem_idx = scratch[NBUF]
    sem_pos = list(scratch[NBUF + 1:NBUF + 1 + NBUF])
    sem_gat = list(scratch[NBUF + 1 + NBUF:NBUF + 1 + 2 * NBUF])
    sem_out = list(scratch[NBUF + 1 + 2 * NBUF:NBUF + 1 + 3 * NBUF])

    sid = lax.axis_index("s")
    wid = sid * NC + lax.axis_index("c")

    @pl.when(sid == 0)
    def _load_pos():
        pltpu.sync_copy(pos_hbm, pos_s)

    d_idx = pltpu.async_copy(tok_hbm.at[pl.ds(wid * ROWS_PER_W, ROWS_PER_W)],
                             idx_v, sem_idx)
    plsc.subcore_barrier()

    d_pos = [None] * NBUF
    d_out = [None] * NBUF
    d_gat = [None] * NBUF

    def start_pos(jw):
        s = jw % NBUF
        if d_out[s] is not None:
            d_out[s].wait()
            d_out[s] = None
        d_pos[s] = pltpu.async_copy(pos_s, bufs[s], sem_pos[s])

    def start_gather(jw):
        s = jw % NBUF
        d_pos[s].wait()
        d_gat[s] = [
            pltpu.async_copy(
                tab_hbm.at[idx_v.at[jw].at[h]],
                bufs[s].at[pl.ds(h * WH, WH)],
                sem_gat[s], add=True)
            for h in range(H)
        ]

    def finish(jw):
        s = jw % NBUF
        for d in d_gat[s]:
            d.wait()
        row = wid * ROWS_PER_W + jw
        d_out[s] = pltpu.async_copy(bufs[s], out_hbm.at[pl.ds(row * W, W)],
                                    sem_out[s])

    for jw in range(NBUF - 1):
        d_pos[jw] = pltpu.async_copy(pos_s, bufs[jw], sem_pos[jw])
    d_idx.wait()
    start_gather(0)

    for j in range(ROWS_PER_W):
        if j + 1 < ROWS_PER_W:
            start_gather(j + 1)
        if j + NBUF - 1 < ROWS_PER_W:
            start_pos(j + NBUF - 1)
        finish(j)

    for s in range(NBUF):
        if d_out[s] is not None:
            d_out[s].wait()


def kernel(tokens, token_embedding, position_embedding, position_indices):
    del position_indices
    tokens3 = tokens.reshape(B, H, WH).astype(jnp.int32)
    mesh = plsc.VectorSubcoreMesh(
        core_axis_name="c", subcore_axis_name="s",
        num_cores=NC, num_subcores=NS,
    )
    out = pl.kernel(
        _body,
        out_type=jax.ShapeDtypeStruct((B * W, D), jnp.float32),
        mesh=mesh,
        scratch_types=[
            pltpu.VMEM((ROWS_PER_W, H, WH), jnp.int32),
            pltpu.VMEM_SHARED((W, D), jnp.float32),
        ] + [pltpu.VMEM((W, D), jnp.float32)] * NBUF
          + [pltpu.SemaphoreType.DMA] * (1 + 3 * NBUF),
    )(token_embedding, tokens3, position_embedding)
    return out.reshape(B, W, D)
